# baseline (speedup 1.0000x reference)
import functools

import jax
import jax.numpy as jnp
from jax import lax
from jax.experimental import pallas as pl
from jax.experimental.pallas import tpu as pltpu
from jax.experimental.pallas import tpu_sc as plsc

B, L, D = 1, 2048, 1024
H = 16
HD = D // H
HALF = HD // 2
I = 2816
THETA = 1000000.0
EPS = 1e-06

BM = 256
BQ = 512
BT = 256
NB = L // BT + 1
LP = NB * BT
BI = 1408
NI = I // BI

_F32 = jnp.float32
_BF16 = jnp.bfloat16


def _qkv_body(h_ref, r_ref, w1_ref, wqkv_ref, bqkv_ref, res_ref, qkv_ref):
    r = h_ref[...] + r_ref[...]
    res_ref[...] = r
    var = jnp.mean(r * r, axis=-1, keepdims=True)
    hn = (r * lax.rsqrt(var + EPS) * w1_ref[...]).astype(_BF16)
    qkv_ref[...] = (
        jnp.dot(hn, wqkv_ref[...], preferred_element_type=_F32) + bqkv_ref[...]
    )


def _qkv_call(h, r, w1, wqkv, bqkv):
    return pl.pallas_call(
        _qkv_body,
        grid=(L // BM,),
        in_specs=[
            pl.BlockSpec((BM, D), lambda i: (i, 0)),
            pl.BlockSpec((BM, D), lambda i: (i, 0)),
            pl.BlockSpec((1, D), lambda i: (0, 0)),
            pl.BlockSpec((D, 3 * D), lambda i: (0, 0)),
            pl.BlockSpec((1, 3 * D), lambda i: (0, 0)),
        ],
        out_specs=[
            pl.BlockSpec((BM, D), lambda i: (i, 0)),
            pl.BlockSpec((BM, 3 * D), lambda i: (i, 0)),
        ],
        out_shape=[
            jax.ShapeDtypeStruct((L, D), _F32),
            jax.ShapeDtypeStruct((L, 3 * D), _F32),
        ],
    )(h, r, w1, wqkv, bqkv)


def _rope(x, c, s):
    x1 = x[:, :HALF]
    x2 = x[:, HALF:]
    return jnp.concatenate([x1 * c - x2 * s, x2 * c + x1 * s], axis=1)


def _attn_body(q_ref, k_ref, v_ref, cq_ref, sq_ref, ck_ref, sk_ref, o_ref):
    i = pl.program_id(1)
    qr = _rope(q_ref[0], cq_ref[...], sq_ref[...]).astype(_BF16)

    def step(j, carry):
        m, den, acc = carry
        kj = _rope(k_ref[0, pl.ds(j * BQ, BQ), :],
                   ck_ref[pl.ds(j * BQ, BQ), :],
                   sk_ref[pl.ds(j * BQ, BQ), :]).astype(_BF16)
        vj = v_ref[0, pl.ds(j * BQ, BQ), :].astype(_BF16)
        s = lax.dot_general(
            qr, kj, (((1,), (1,)), ((), ())), preferred_element_type=_F32
        ) * (1.0 / (HD ** 0.5))
        row = lax.broadcasted_iota(jnp.int32, (BQ, BQ), 0) + i * BQ
        col = lax.broadcasted_iota(jnp.int32, (BQ, BQ), 1) + j * BQ
        s = jnp.where(col <= row, s, _F32(-1e9))
        m_new = jnp.maximum(m, jnp.max(s, axis=1, keepdims=True))
        p = jnp.exp(s - m_new)
        corr = jnp.exp(m - m_new)
        den_new = den * corr + jnp.sum(p, axis=1, keepdims=True)
        acc_new = acc * corr + jnp.dot(
            p.astype(_BF16), vj, preferred_element_type=_F32
        )
        return m_new, den_new, acc_new

    m0 = jnp.full((BQ, 1), -jnp.inf, _F32)
    d0 = jnp.zeros((BQ, 1), _F32)
    a0 = jnp.zeros((BQ, HD), _F32)
    m, den, acc = lax.fori_loop(0, i + 1, step, (m0, d0, a0))
    o_ref[0] = acc / den


def _attn_call(q3, k3, v3, cos, sin):
    return pl.pallas_call(
        _attn_body,
        grid=(H, L // BQ),
        in_specs=[
            pl.BlockSpec((1, BQ, HD), lambda h, i: (h, i, 0)),
            pl.BlockSpec((1, L, HD), lambda h, i: (h, 0, 0)),
            pl.BlockSpec((1, L, HD), lambda h, i: (h, 0, 0)),
            pl.BlockSpec((BQ, HALF), lambda h, i: (i, 0)),
            pl.BlockSpec((BQ, HALF), lambda h, i: (i, 0)),
            pl.BlockSpec((L, HALF), lambda h, i: (0, 0)),
            pl.BlockSpec((L, HALF), lambda h, i: (0, 0)),
        ],
        out_specs=pl.BlockSpec((1, BQ, HD), lambda h, i: (h, i, 0)),
        out_shape=jax.ShapeDtypeStruct((H, L, HD), _F32),
    )(q3, k3, v3, cos, sin, cos, sin)


def _out_body(a_ref, wo_ref, r_ref, w2_ref, res2_ref, h2_ref):
    a = a_ref[...].astype(_BF16)
    r2 = jnp.dot(a, wo_ref[...], preferred_element_type=_F32) + r_ref[...]
    res2_ref[...] = r2
    var = jnp.mean(r2 * r2, axis=-1, keepdims=True)
    h2_ref[...] = (r2 * lax.rsqrt(var + EPS) * w2_ref[...]).astype(_BF16)


def _out_call(attn, wo, res1, w2):
    return pl.pallas_call(
        _out_body,
        grid=(L // BM,),
        in_specs=[
            pl.BlockSpec((BM, D), lambda i: (i, 0)),
            pl.BlockSpec((D, D), lambda i: (0, 0)),
            pl.BlockSpec((BM, D), lambda i: (i, 0)),
            pl.BlockSpec((1, D), lambda i: (0, 0)),
        ],
        out_specs=[
            pl.BlockSpec((BM, D), lambda i: (i, 0)),
            pl.BlockSpec((BM, D), lambda i: (i, 0)),
        ],
        out_shape=[
            jax.ShapeDtypeStruct((L, D), _F32),
            jax.ShapeDtypeStruct((L, D), _BF16),
        ],
    )(attn, wo, res1, w2)


def _moe_body(eid_ref, x_ref, wg_ref, wu_ref, wd_ref, o_ref):
    del eid_ref
    x = x_ref[...]
    g = jnp.dot(x, wg_ref[0], preferred_element_type=_F32)
    u = jnp.dot(x, wu_ref[0], preferred_element_type=_F32)
    a = (g * lax.logistic(g) * u).astype(_BF16)
    o_ref[...] = jnp.dot(a, wd_ref[0], preferred_element_type=_F32).astype(_BF16)


def _moe_call(block_expert, xp, WG, WU, WD):
    grid_spec = pltpu.PrefetchScalarGridSpec(
        num_scalar_prefetch=1,
        grid=(NB,),
        in_specs=[
            pl.BlockSpec((BT, D), lambda b, eid: (b, 0)),
            pl.BlockSpec((1, D, I), lambda b, eid: (eid[b], 0, 0)),
            pl.BlockSpec((1, D, I), lambda b, eid: (eid[b], 0, 0)),
            pl.BlockSpec((1, I, D), lambda b, eid: (eid[b], 0, 0)),
        ],
        out_specs=pl.BlockSpec((BT, D), lambda b, eid: (b, 0)),
    )
    return pl.pallas_call(
        _moe_body,
        grid_spec=grid_spec,
        out_shape=jax.ShapeDtypeStruct((LP, D), _BF16),
        compiler_params=pltpu.CompilerParams(
            dimension_semantics=("arbitrary",),
        ),
    )(block_expert, xp, WG, WU, WD)


def _sc_gather(table, idx):
    v_rows, d = table.shape
    n = idx.shape[0]
    info = plsc.get_sparse_core_info()
    nw = info.num_cores * info.num_subcores
    bpw = n // nw
    mesh = plsc.VectorSubcoreMesh(core_axis_name="c", subcore_axis_name="s")

    @functools.partial(
        pl.kernel,
        mesh=mesh,
        out_type=jax.ShapeDtypeStruct((n, d), jnp.int32),
        scratch_types=[
            pltpu.VMEM((bpw,), jnp.int32),
            pltpu.VMEM((bpw, d), jnp.int32),
            pltpu.SemaphoreType.DMA,
        ],
    )
    def k(table_hbm, idx_hbm, out_hbm, idx_v, rows_v, sem):
        wid = lax.axis_index("s") * info.num_cores + lax.axis_index("c")
        base = wid * bpw
        pltpu.sync_copy(idx_hbm.at[pl.ds(base, bpw)], idx_v)
        pltpu.async_copy(table_hbm.at[idx_v], rows_v, sem).wait()
        pltpu.sync_copy(rows_v, out_hbm.at[pl.ds(base, bpw)])

    return k(table, idx)


def _to_i32(x_bf16):
    return lax.bitcast_convert_type(
        x_bf16.reshape(x_bf16.shape[0], -1, 2), jnp.int32)


def _from_i32(x_i32):
    return lax.bitcast_convert_type(x_i32, _BF16).reshape(x_i32.shape[0], -1)


def kernel(positions, hidden_states, residual, gen_token_mask, rms1_w, rms2_w,
           wq, bq, wk, bk, wv, bv, wo, wg, wu, wd, gwg, gwu, gwd):
    h = hidden_states.reshape(L, D)
    r = residual.reshape(L, D)
    wqkv = jnp.concatenate([wq, wk, wv], axis=1).astype(_BF16)
    bqkv = jnp.concatenate([bq, bk, bv])[None, :]

    res1, qkv = _qkv_call(h, r, rms1_w[None, :], wqkv, bqkv)

    qkv3 = qkv.reshape(L, 3, H, HD).transpose(1, 2, 0, 3)

    pos = positions.reshape(-1).astype(_F32)
    inv_freq = 1.0 / (THETA ** (jnp.arange(HALF, dtype=_F32) / HALF))
    freqs = pos[:, None] * inv_freq[None, :]
    cos = jnp.cos(freqs)
    sin = jnp.sin(freqs)

    attn = _attn_call(qkv3[0], qkv3[1], qkv3[2], cos, sin)
    attn2 = attn.transpose(1, 0, 2).reshape(L, D)
    res2, h2 = _out_call(attn2, wo.astype(_BF16), res1, rms2_w[None, :])

    mi = gen_token_mask.reshape(-1).astype(jnp.int32)
    cg_cum = jnp.cumsum(mi)
    cu_cum = jnp.cumsum(1 - mi)
    cu = cu_cum[-1]
    nbu = (cu + BT - 1) // BT
    gen_start = nbu * BT
    dest = jnp.where(mi == 1, gen_start + cg_cum - 1, cu_cum - 1).astype(jnp.int32)
    perm = jnp.zeros((LP,), jnp.int32).at[dest].set(
        jnp.arange(L, dtype=jnp.int32))
    block_expert = (jnp.arange(NB, dtype=jnp.int32) >= nbu).astype(jnp.int32)

    xp = _from_i32(_sc_gather(_to_i32(h2), perm))
    WG = jnp.stack([wg, gwg]).astype(_BF16)
    WU = jnp.stack([wu, gwu]).astype(_BF16)
    WD = jnp.stack([wd, gwd]).astype(_BF16)
    mo = _moe_call(block_expert, xp, WG, WU, WD)
    out = _from_i32(_sc_gather(_to_i32(mo), dest)).astype(_F32)

    return (out.reshape(B, L, D), res2.reshape(B, L, D))


# speedup vs baseline: 1.0864x; 1.0864x over previous
import functools

import jax
import jax.numpy as jnp
from jax import lax
from jax.experimental import pallas as pl
from jax.experimental.pallas import tpu as pltpu
from jax.experimental.pallas import tpu_sc as plsc

B, L, D = 1, 2048, 1024
H = 16
HD = D // H
HALF = HD // 2
I = 2816
THETA = 1000000.0
EPS = 1e-06

BM = 256
BQ = 512
BT = 256
NB = L // BT + 1
LP = NB * BT
BI = 1408
NI = I // BI

_F32 = jnp.float32
_BF16 = jnp.bfloat16


def _qkv_body(h_ref, r_ref, w1_ref, wqkv_ref, bqkv_ref, res_ref, qkv_ref):
    r = h_ref[...] + r_ref[...]
    res_ref[...] = r
    var = jnp.mean(r * r, axis=-1, keepdims=True)
    hn = (r * lax.rsqrt(var + EPS) * w1_ref[...]).astype(_BF16)
    qkv_ref[...] = (
        jnp.dot(hn, wqkv_ref[...], preferred_element_type=_F32) + bqkv_ref[...]
    )


def _qkv_call(h, r, w1, wqkv, bqkv):
    return pl.pallas_call(
        _qkv_body,
        grid=(L // BM,),
        in_specs=[
            pl.BlockSpec((BM, D), lambda i: (i, 0)),
            pl.BlockSpec((BM, D), lambda i: (i, 0)),
            pl.BlockSpec((1, D), lambda i: (0, 0)),
            pl.BlockSpec((D, 3 * D), lambda i: (0, 0)),
            pl.BlockSpec((1, 3 * D), lambda i: (0, 0)),
        ],
        out_specs=[
            pl.BlockSpec((BM, D), lambda i: (i, 0)),
            pl.BlockSpec((BM, 3 * D), lambda i: (i, 0)),
        ],
        out_shape=[
            jax.ShapeDtypeStruct((L, D), _F32),
            jax.ShapeDtypeStruct((L, 3 * D), _F32),
        ],
    )(h, r, w1, wqkv, bqkv)


def _rope(x, c, s):
    x1 = x[:, :HALF]
    x2 = x[:, HALF:]
    return jnp.concatenate([x1 * c - x2 * s, x2 * c + x1 * s], axis=1)


def _attn_body(q_ref, k_ref, v_ref, cq_ref, sq_ref, ck_ref, sk_ref, o_ref,
               qr_ref, kr_ref, m_ref, d_ref, a_ref):
    i = pl.program_id(1)
    j = pl.program_id(2)
    nkv = pl.num_programs(2)

    @pl.when(j == 0)
    def _():
        qr_ref[...] = (
            _rope(q_ref[0], cq_ref[...], sq_ref[...]) * (1.0 / (HD ** 0.5))
        ).astype(_BF16)
        m_ref[...] = jnp.full_like(m_ref[...], -1e30)
        d_ref[...] = jnp.zeros_like(d_ref[...])
        a_ref[...] = jnp.zeros_like(a_ref[...])

    @pl.when(j == i)
    def _():
        kr_ref[pl.ds(j * BQ, BQ), :] = _rope(
            k_ref[0], ck_ref[...], sk_ref[...]).astype(_BF16)

    @pl.when(j <= i)
    def _():
        qr = qr_ref[...]
        kr = kr_ref[pl.ds(j * BQ, BQ), :]
        s = lax.dot_general(
            qr, kr, (((1,), (1,)), ((), ())), preferred_element_type=_F32)
        row = lax.broadcasted_iota(jnp.int32, (BQ, BQ), 0) + i * BQ
        col = lax.broadcasted_iota(jnp.int32, (BQ, BQ), 1) + j * BQ
        s = jnp.where(col <= row, s, _F32(-1e30))
        m_prev = m_ref[...]
        m_new = jnp.maximum(m_prev, jnp.max(s, axis=1, keepdims=True))
        m_ref[...] = m_new
        p = jnp.exp(s - m_new)
        corr = jnp.exp(m_prev - m_new)
        d_ref[...] = d_ref[...] * corr + jnp.sum(p, axis=1, keepdims=True)
        a_ref[...] = a_ref[...] * corr + jnp.dot(
            p.astype(_BF16), v_ref[0].astype(_BF16),
            preferred_element_type=_F32)

    @pl.when(j == nkv - 1)
    def _():
        o_ref[0] = a_ref[...] / d_ref[...]


def _attn_call(q3, k3, v3, cos, sin):
    return pl.pallas_call(
        _attn_body,
        grid=(H, L // BQ, L // BQ),
        in_specs=[
            pl.BlockSpec((1, BQ, HD), lambda h, i, j: (h, i, 0)),
            pl.BlockSpec((1, BQ, HD),
                         lambda h, i, j: (h, jnp.minimum(i, j), 0)),
            pl.BlockSpec((1, BQ, HD),
                         lambda h, i, j: (h, jnp.minimum(i, j), 0)),
            pl.BlockSpec((BQ, HALF), lambda h, i, j: (i, 0)),
            pl.BlockSpec((BQ, HALF), lambda h, i, j: (i, 0)),
            pl.BlockSpec((BQ, HALF), lambda h, i, j: (jnp.minimum(i, j), 0)),
            pl.BlockSpec((BQ, HALF), lambda h, i, j: (jnp.minimum(i, j), 0)),
        ],
        out_specs=pl.BlockSpec((1, BQ, HD), lambda h, i, j: (h, i, 0)),
        out_shape=jax.ShapeDtypeStruct((H, L, HD), _F32),
        scratch_shapes=[
            pltpu.VMEM((BQ, HD), _BF16),
            pltpu.VMEM((L, HD), _BF16),
            pltpu.VMEM((BQ, 1), _F32),
            pltpu.VMEM((BQ, 1), _F32),
            pltpu.VMEM((BQ, HD), _F32),
        ],
        compiler_params=pltpu.CompilerParams(
            dimension_semantics=("arbitrary", "arbitrary", "arbitrary"),
        ),
    )(q3, k3, v3, cos, sin, cos, sin)


def _out_body(a_ref, wo_ref, r_ref, w2_ref, res2_ref, h2_ref):
    a = a_ref[...].astype(_BF16)
    r2 = jnp.dot(a, wo_ref[...], preferred_element_type=_F32) + r_ref[...]
    res2_ref[...] = r2
    var = jnp.mean(r2 * r2, axis=-1, keepdims=True)
    h2_ref[...] = r2 * lax.rsqrt(var + EPS) * w2_ref[...]


def _out_call(attn, wo, res1, w2):
    return pl.pallas_call(
        _out_body,
        grid=(L // BM,),
        in_specs=[
            pl.BlockSpec((BM, D), lambda i: (i, 0)),
            pl.BlockSpec((D, D), lambda i: (0, 0)),
            pl.BlockSpec((BM, D), lambda i: (i, 0)),
            pl.BlockSpec((1, D), lambda i: (0, 0)),
        ],
        out_specs=[
            pl.BlockSpec((BM, D), lambda i: (i, 0)),
            pl.BlockSpec((BM, D), lambda i: (i, 0)),
        ],
        out_shape=[
            jax.ShapeDtypeStruct((L, D), _F32),
            jax.ShapeDtypeStruct((L, D), _F32),
        ],
    )(attn, wo, res1, w2)


def _moe_body(eid_ref, x_ref, wg_ref, wu_ref, wd_ref, o_ref):
    del eid_ref
    x = x_ref[...].astype(_BF16)
    g = jnp.dot(x, wg_ref[0], preferred_element_type=_F32)
    u = jnp.dot(x, wu_ref[0], preferred_element_type=_F32)
    a = (g * lax.logistic(g) * u).astype(_BF16)
    o_ref[...] = jnp.dot(a, wd_ref[0], preferred_element_type=_F32)


def _moe_call(block_expert, xp, WG, WU, WD):
    grid_spec = pltpu.PrefetchScalarGridSpec(
        num_scalar_prefetch=1,
        grid=(NB,),
        in_specs=[
            pl.BlockSpec((BT, D), lambda b, eid: (b, 0)),
            pl.BlockSpec((1, D, I), lambda b, eid: (eid[b], 0, 0)),
            pl.BlockSpec((1, D, I), lambda b, eid: (eid[b], 0, 0)),
            pl.BlockSpec((1, I, D), lambda b, eid: (eid[b], 0, 0)),
        ],
        out_specs=pl.BlockSpec((BT, D), lambda b, eid: (b, 0)),
    )
    return pl.pallas_call(
        _moe_body,
        grid_spec=grid_spec,
        out_shape=jax.ShapeDtypeStruct((LP, D), _F32),
        compiler_params=pltpu.CompilerParams(
            dimension_semantics=("arbitrary",),
        ),
    )(block_expert, xp, WG, WU, WD)


def _sc_gather(table, idx):
    v_rows, d = table.shape
    n = idx.shape[0]
    info = plsc.get_sparse_core_info()
    nw = info.num_cores * info.num_subcores
    bpw = n // nw
    mesh = plsc.VectorSubcoreMesh(core_axis_name="c", subcore_axis_name="s")

    @functools.partial(
        pl.kernel,
        mesh=mesh,
        out_type=jax.ShapeDtypeStruct((n, d), _F32),
        scratch_types=[
            pltpu.VMEM((bpw,), jnp.int32),
            pltpu.VMEM((bpw, d), _F32),
            pltpu.SemaphoreType.DMA,
        ],
    )
    def k(table_hbm, idx_hbm, out_hbm, idx_v, rows_v, sem):
        wid = lax.axis_index("s") * info.num_cores + lax.axis_index("c")
        base = wid * bpw
        pltpu.sync_copy(idx_hbm.at[pl.ds(base, bpw)], idx_v)
        pltpu.async_copy(table_hbm.at[idx_v], rows_v, sem).wait()
        pltpu.sync_copy(rows_v, out_hbm.at[pl.ds(base, bpw)])

    return k(table, idx)


def kernel(positions, hidden_states, residual, gen_token_mask, rms1_w, rms2_w,
           wq, bq, wk, bk, wv, bv, wo, wg, wu, wd, gwg, gwu, gwd):
    h = hidden_states.reshape(L, D)
    r = residual.reshape(L, D)
    wqkv = jnp.concatenate([wq, wk, wv], axis=1).astype(_BF16)
    bqkv = jnp.concatenate([bq, bk, bv])[None, :]

    res1, qkv = _qkv_call(h, r, rms1_w[None, :], wqkv, bqkv)

    qkv3 = qkv.reshape(L, 3, H, HD).transpose(1, 2, 0, 3)

    pos = positions.reshape(-1).astype(_F32)
    inv_freq = 1.0 / (THETA ** (jnp.arange(HALF, dtype=_F32) / HALF))
    freqs = pos[:, None] * inv_freq[None, :]
    cos = jnp.cos(freqs)
    sin = jnp.sin(freqs)

    attn = _attn_call(qkv3[0], qkv3[1], qkv3[2], cos, sin)
    attn2 = attn.transpose(1, 0, 2).reshape(L, D)
    res2, h2 = _out_call(attn2, wo.astype(_BF16), res1, rms2_w[None, :])

    mi = gen_token_mask.reshape(-1).astype(jnp.int32)
    cg_cum = jnp.cumsum(mi)
    cu_cum = jnp.cumsum(1 - mi)
    cu = cu_cum[-1]
    nbu = (cu + BT - 1) // BT
    gen_start = nbu * BT
    dest = jnp.where(mi == 1, gen_start + cg_cum - 1, cu_cum - 1).astype(jnp.int32)
    perm = jnp.zeros((LP,), jnp.int32).at[dest].set(
        jnp.arange(L, dtype=jnp.int32))
    block_expert = (jnp.arange(NB, dtype=jnp.int32) >= nbu).astype(jnp.int32)

    xp = _sc_gather(h2, perm)
    WG = jnp.stack([wg, gwg]).astype(_BF16)
    WU = jnp.stack([wu, gwu]).astype(_BF16)
    WD = jnp.stack([wd, gwd]).astype(_BF16)
    mo = _moe_call(block_expert, xp, WG, WU, WD)
    out = _sc_gather(mo, dest)

    return (out.reshape(B, L, D), res2.reshape(B, L, D))


# speedup vs baseline: 1.2557x; 1.1559x over previous
import functools

import jax
import jax.numpy as jnp
from jax import lax
from jax.experimental import pallas as pl
from jax.experimental.pallas import tpu as pltpu
from jax.experimental.pallas import tpu_sc as plsc

B, L, D = 1, 2048, 1024
H = 16
HD = D // H
HALF = HD // 2
I = 2816
THETA = 1000000.0
EPS = 1e-06

BM = 256
BQ = 512
BT = 256
NB = L // BT + 1
LP = NB * BT
BI = 1408
NI = I // BI

_F32 = jnp.float32
_BF16 = jnp.bfloat16


def _qkv_body(h_ref, r_ref, w1_ref, wqkv_ref, bqkv_ref, res_ref, qkv_ref):
    r = h_ref[...] + r_ref[...]
    res_ref[...] = r
    var = jnp.mean(r * r, axis=-1, keepdims=True)
    hn = (r * lax.rsqrt(var + EPS) * w1_ref[...]).astype(_BF16)
    qkv_ref[...] = (
        jnp.dot(hn, wqkv_ref[...], preferred_element_type=_F32) + bqkv_ref[...]
    )


def _qkv_call(h, r, w1, wqkv, bqkv):
    return pl.pallas_call(
        _qkv_body,
        grid=(L // BM,),
        in_specs=[
            pl.BlockSpec((BM, D), lambda i: (i, 0)),
            pl.BlockSpec((BM, D), lambda i: (i, 0)),
            pl.BlockSpec((1, D), lambda i: (0, 0)),
            pl.BlockSpec((D, 3 * D), lambda i: (0, 0)),
            pl.BlockSpec((1, 3 * D), lambda i: (0, 0)),
        ],
        out_specs=[
            pl.BlockSpec((BM, D), lambda i: (i, 0)),
            pl.BlockSpec((BM, 3 * D), lambda i: (i, 0)),
        ],
        out_shape=[
            jax.ShapeDtypeStruct((L, D), _F32),
            jax.ShapeDtypeStruct((L, 3 * D), _F32),
        ],
    )(h, r, w1, wqkv, bqkv)


def _rope(x, c, s):
    x1 = x[:, :HALF]
    x2 = x[:, HALF:]
    return jnp.concatenate([x1 * c - x2 * s, x2 * c + x1 * s], axis=1)


def _attn_body(q_ref, k_ref, v_ref, cq_ref, sq_ref, ck_ref, sk_ref, bias_ref,
               o_ref):
    i = pl.program_id(1)
    qr = (
        _rope(q_ref[0], cq_ref[...], sq_ref[...]) * (1.0 / (HD ** 0.5))
    ).astype(_BF16)

    def step(j, carry):
        m, den, acc = carry
        kj = _rope(k_ref[0, pl.ds(j * BQ, BQ), :],
                   ck_ref[pl.ds(j * BQ, BQ), :],
                   sk_ref[pl.ds(j * BQ, BQ), :]).astype(_BF16)
        vj = v_ref[0, pl.ds(j * BQ, BQ), :].astype(_BF16)
        s = lax.dot_general(
            qr, kj, (((1,), (1,)), ((), ())), preferred_element_type=_F32
        )
        s = lax.cond(j == i, lambda x: x + bias_ref[...], lambda x: x, s)
        m_new = jnp.maximum(m, jnp.max(s, axis=1, keepdims=True))
        p = jnp.exp(s - m_new)
        corr = jnp.exp(m - m_new)
        den_new = den * corr + jnp.sum(p, axis=1, keepdims=True)
        acc_new = acc * corr + jnp.dot(
            p.astype(_BF16), vj, preferred_element_type=_F32
        )
        return m_new, den_new, acc_new

    m0 = jnp.full((BQ, 1), -1e30, _F32)
    d0 = jnp.zeros((BQ, 1), _F32)
    a0 = jnp.zeros((BQ, HD), _F32)
    m, den, acc = lax.fori_loop(0, i + 1, step, (m0, d0, a0))
    o_ref[0] = acc / den


def _attn_call(q3, k3, v3, cos, sin):
    r = lax.broadcasted_iota(jnp.int32, (BQ, BQ), 0)
    c = lax.broadcasted_iota(jnp.int32, (BQ, BQ), 1)
    bias = jnp.where(c <= r, _F32(0), _F32(-1e30))
    return pl.pallas_call(
        _attn_body,
        grid=(H, L // BQ),
        in_specs=[
            pl.BlockSpec((1, BQ, HD), lambda h, i: (h, i, 0)),
            pl.BlockSpec((1, L, HD), lambda h, i: (h, 0, 0)),
            pl.BlockSpec((1, L, HD), lambda h, i: (h, 0, 0)),
            pl.BlockSpec((BQ, HALF), lambda h, i: (i, 0)),
            pl.BlockSpec((BQ, HALF), lambda h, i: (i, 0)),
            pl.BlockSpec((L, HALF), lambda h, i: (0, 0)),
            pl.BlockSpec((L, HALF), lambda h, i: (0, 0)),
            pl.BlockSpec((BQ, BQ), lambda h, i: (0, 0)),
        ],
        out_specs=pl.BlockSpec((1, BQ, HD), lambda h, i: (h, i, 0)),
        out_shape=jax.ShapeDtypeStruct((H, L, HD), _F32),
    )(q3, k3, v3, cos, sin, cos, sin, bias)


def _out_body(a_ref, wo_ref, r_ref, w2_ref, res2_ref, h2_ref):
    a = a_ref[...].astype(_BF16)
    r2 = jnp.dot(a, wo_ref[...], preferred_element_type=_F32) + r_ref[...]
    res2_ref[...] = r2
    var = jnp.mean(r2 * r2, axis=-1, keepdims=True)
    h2_ref[...] = r2 * lax.rsqrt(var + EPS) * w2_ref[...]


def _out_call(attn, wo, res1, w2):
    return pl.pallas_call(
        _out_body,
        grid=(L // BM,),
        in_specs=[
            pl.BlockSpec((BM, D), lambda i: (i, 0)),
            pl.BlockSpec((D, D), lambda i: (0, 0)),
            pl.BlockSpec((BM, D), lambda i: (i, 0)),
            pl.BlockSpec((1, D), lambda i: (0, 0)),
        ],
        out_specs=[
            pl.BlockSpec((BM, D), lambda i: (i, 0)),
            pl.BlockSpec((BM, D), lambda i: (i, 0)),
        ],
        out_shape=[
            jax.ShapeDtypeStruct((L, D), _F32),
            jax.ShapeDtypeStruct((L, D), _F32),
        ],
    )(attn, wo, res1, w2)


def _moe_body(eid_ref, x_ref, wg_ref, wu_ref, wd_ref, o_ref):
    del eid_ref
    x = x_ref[...].astype(_BF16)
    g = jnp.dot(x, wg_ref[0], preferred_element_type=_F32)
    u = jnp.dot(x, wu_ref[0], preferred_element_type=_F32)
    a = (g * lax.logistic(g) * u).astype(_BF16)
    o_ref[...] = jnp.dot(a, wd_ref[0], preferred_element_type=_F32)


def _moe_call(block_expert, xp, WG, WU, WD):
    grid_spec = pltpu.PrefetchScalarGridSpec(
        num_scalar_prefetch=1,
        grid=(NB,),
        in_specs=[
            pl.BlockSpec((BT, D), lambda b, eid: (b, 0)),
            pl.BlockSpec((1, D, I), lambda b, eid: (eid[b], 0, 0)),
            pl.BlockSpec((1, D, I), lambda b, eid: (eid[b], 0, 0)),
            pl.BlockSpec((1, I, D), lambda b, eid: (eid[b], 0, 0)),
        ],
        out_specs=pl.BlockSpec((BT, D), lambda b, eid: (b, 0)),
    )
    return pl.pallas_call(
        _moe_body,
        grid_spec=grid_spec,
        out_shape=jax.ShapeDtypeStruct((LP, D), _F32),
        compiler_params=pltpu.CompilerParams(
            dimension_semantics=("arbitrary",),
        ),
    )(block_expert, xp, WG, WU, WD)


def _sc_gather(table, idx):
    v_rows, d = table.shape
    n = idx.shape[0]
    info = plsc.get_sparse_core_info()
    nw = info.num_cores * info.num_subcores
    bpw = n // nw
    mesh = plsc.VectorSubcoreMesh(core_axis_name="c", subcore_axis_name="s")

    @functools.partial(
        pl.kernel,
        mesh=mesh,
        out_type=jax.ShapeDtypeStruct((n, d), _F32),
        scratch_types=[
            pltpu.VMEM((bpw,), jnp.int32),
            pltpu.VMEM((bpw, d), _F32),
            pltpu.SemaphoreType.DMA,
        ],
    )
    def k(table_hbm, idx_hbm, out_hbm, idx_v, rows_v, sem):
        wid = lax.axis_index("s") * info.num_cores + lax.axis_index("c")
        base = wid * bpw
        pltpu.sync_copy(idx_hbm.at[pl.ds(base, bpw)], idx_v)
        pltpu.async_copy(table_hbm.at[idx_v], rows_v, sem).wait()
        pltpu.sync_copy(rows_v, out_hbm.at[pl.ds(base, bpw)])

    return k(table, idx)


def kernel(positions, hidden_states, residual, gen_token_mask, rms1_w, rms2_w,
           wq, bq, wk, bk, wv, bv, wo, wg, wu, wd, gwg, gwu, gwd):
    h = hidden_states.reshape(L, D)
    r = residual.reshape(L, D)
    wqkv = jnp.concatenate([wq, wk, wv], axis=1).astype(_BF16)
    bqkv = jnp.concatenate([bq, bk, bv])[None, :]

    res1, qkv = _qkv_call(h, r, rms1_w[None, :], wqkv, bqkv)

    qkv3 = qkv.reshape(L, 3, H, HD).transpose(1, 2, 0, 3)

    pos = positions.reshape(-1).astype(_F32)
    inv_freq = 1.0 / (THETA ** (jnp.arange(HALF, dtype=_F32) / HALF))
    freqs = pos[:, None] * inv_freq[None, :]
    cos = jnp.cos(freqs)
    sin = jnp.sin(freqs)

    attn = _attn_call(qkv3[0], qkv3[1], qkv3[2], cos, sin)
    attn2 = attn.transpose(1, 0, 2).reshape(L, D)
    res2, h2 = _out_call(attn2, wo.astype(_BF16), res1, rms2_w[None, :])

    mi = gen_token_mask.reshape(-1).astype(jnp.int32)
    cg_cum = jnp.cumsum(mi)
    cu_cum = jnp.cumsum(1 - mi)
    cu = cu_cum[-1]
    nbu = (cu + BT - 1) // BT
    gen_start = nbu * BT
    dest = jnp.where(mi == 1, gen_start + cg_cum - 1, cu_cum - 1).astype(jnp.int32)
    perm = jnp.zeros((LP,), jnp.int32).at[dest].set(
        jnp.arange(L, dtype=jnp.int32))
    block_expert = (jnp.arange(NB, dtype=jnp.int32) >= nbu).astype(jnp.int32)

    xp = _sc_gather(h2, perm)
    WG = jnp.stack([wg, gwg]).astype(_BF16)
    WU = jnp.stack([wu, gwu]).astype(_BF16)
    WD = jnp.stack([wd, gwd]).astype(_BF16)
    mo = _moe_call(block_expert, xp, WG, WU, WD)
    out = _sc_gather(mo, dest)

    return (out.reshape(B, L, D), res2.reshape(B, L, D))


# speedup vs baseline: 1.4393x; 1.1462x over previous
import functools

import jax
import jax.numpy as jnp
from jax import lax
from jax.experimental import pallas as pl
from jax.experimental.pallas import tpu as pltpu
from jax.experimental.pallas import tpu_sc as plsc

B, L, D = 1, 2048, 1024
H = 16
HD = D // H
HALF = HD // 2
I = 2816
THETA = 1000000.0
EPS = 1e-06

BM = 256
BQ = 512
BT = 256
NB = L // BT + 1
LP = NB * BT
BI = 1408
NI = I // BI

_F32 = jnp.float32
_BF16 = jnp.bfloat16


def _qkv_body(h_ref, r_ref, w1_ref, wqkv_ref, bqkv_ref, res_ref, qkv_ref):
    r = h_ref[...] + r_ref[...]
    res_ref[...] = r
    var = jnp.mean(r * r, axis=-1, keepdims=True)
    hn = (r * lax.rsqrt(var + EPS) * w1_ref[...]).astype(_BF16)
    qkv_ref[...] = (
        jnp.dot(hn, wqkv_ref[...], preferred_element_type=_F32) + bqkv_ref[...]
    )


def _qkv_call(h, r, w1, wqkv, bqkv):
    return pl.pallas_call(
        _qkv_body,
        grid=(L // BM,),
        in_specs=[
            pl.BlockSpec((BM, D), lambda i: (i, 0)),
            pl.BlockSpec((BM, D), lambda i: (i, 0)),
            pl.BlockSpec((1, D), lambda i: (0, 0)),
            pl.BlockSpec((D, 3 * D), lambda i: (0, 0)),
            pl.BlockSpec((1, 3 * D), lambda i: (0, 0)),
        ],
        out_specs=[
            pl.BlockSpec((BM, D), lambda i: (i, 0)),
            pl.BlockSpec((BM, 3 * D), lambda i: (i, 0)),
        ],
        out_shape=[
            jax.ShapeDtypeStruct((L, D), _F32),
            jax.ShapeDtypeStruct((L, 3 * D), _F32),
        ],
    )(h, r, w1, wqkv, bqkv)


def _rope(x, c, s):
    x1 = x[:, :HALF]
    x2 = x[:, HALF:]
    return jnp.concatenate([x1 * c - x2 * s, x2 * c + x1 * s], axis=1)


def _attn_body(q_ref, k_ref, v_ref, cq_ref, sq_ref, ck_ref, sk_ref, o_ref):
    i = pl.program_id(1)
    qr = _rope(q_ref[0], cq_ref[...], sq_ref[...]).astype(_BF16)

    def step(j, carry):
        m, den, acc = carry
        kj = _rope(k_ref[0, pl.ds(j * BQ, BQ), :],
                   ck_ref[pl.ds(j * BQ, BQ), :],
                   sk_ref[pl.ds(j * BQ, BQ), :]).astype(_BF16)
        vj = v_ref[0, pl.ds(j * BQ, BQ), :].astype(_BF16)
        s = lax.dot_general(
            qr, kj, (((1,), (1,)), ((), ())), preferred_element_type=_F32
        ) * (1.0 / (HD ** 0.5))
        row = lax.broadcasted_iota(jnp.int32, (BQ, BQ), 0) + i * BQ
        col = lax.broadcasted_iota(jnp.int32, (BQ, BQ), 1) + j * BQ
        s = jnp.where(col <= row, s, _F32(-1e9))
        m_new = jnp.maximum(m, jnp.max(s, axis=1, keepdims=True))
        p = jnp.exp(s - m_new)
        corr = jnp.exp(m - m_new)
        den_new = den * corr + jnp.sum(p, axis=1, keepdims=True)
        acc_new = acc * corr + jnp.dot(
            p.astype(_BF16), vj, preferred_element_type=_F32
        )
        return m_new, den_new, acc_new

    m0 = jnp.full((BQ, 1), -jnp.inf, _F32)
    d0 = jnp.zeros((BQ, 1), _F32)
    a0 = jnp.zeros((BQ, HD), _F32)
    m, den, acc = lax.fori_loop(0, i + 1, step, (m0, d0, a0))
    o_ref[0] = acc / den


def _attn_call(q3, k3, v3, cos, sin):
    return pl.pallas_call(
        _attn_body,
        grid=(H, L // BQ),
        in_specs=[
            pl.BlockSpec((1, BQ, HD), lambda h, i: (h, i, 0)),
            pl.BlockSpec((1, L, HD), lambda h, i: (h, 0, 0)),
            pl.BlockSpec((1, L, HD), lambda h, i: (h, 0, 0)),
            pl.BlockSpec((BQ, HALF), lambda h, i: (i, 0)),
            pl.BlockSpec((BQ, HALF), lambda h, i: (i, 0)),
            pl.BlockSpec((L, HALF), lambda h, i: (0, 0)),
            pl.BlockSpec((L, HALF), lambda h, i: (0, 0)),
        ],
        out_specs=pl.BlockSpec((1, BQ, HD), lambda h, i: (h, i, 0)),
        out_shape=jax.ShapeDtypeStruct((H, L, HD), _F32),
    )(q3, k3, v3, cos, sin, cos, sin)


def _out_body(a_ref, wo_ref, r_ref, w2_ref, res2_ref, h2_ref):
    a = a_ref[...].astype(_BF16)
    r2 = jnp.dot(a, wo_ref[...], preferred_element_type=_F32) + r_ref[...]
    res2_ref[...] = r2
    var = jnp.mean(r2 * r2, axis=-1, keepdims=True)
    h2_ref[...] = r2 * lax.rsqrt(var + EPS) * w2_ref[...]


def _out_call(attn, wo, res1, w2):
    return pl.pallas_call(
        _out_body,
        grid=(L // BM,),
        in_specs=[
            pl.BlockSpec((BM, D), lambda i: (i, 0)),
            pl.BlockSpec((D, D), lambda i: (0, 0)),
            pl.BlockSpec((BM, D), lambda i: (i, 0)),
            pl.BlockSpec((1, D), lambda i: (0, 0)),
        ],
        out_specs=[
            pl.BlockSpec((BM, D), lambda i: (i, 0)),
            pl.BlockSpec((BM, D), lambda i: (i, 0)),
        ],
        out_shape=[
            jax.ShapeDtypeStruct((L, D), _F32),
            jax.ShapeDtypeStruct((L, D), _F32),
        ],
    )(attn, wo, res1, w2)


def _moe_body(eid_ref, x_ref, wg_ref, wu_ref, wd_ref, o_ref):
    del eid_ref
    x = x_ref[...].astype(_BF16)
    g = jnp.dot(x, wg_ref[0], preferred_element_type=_F32)
    u = jnp.dot(x, wu_ref[0], preferred_element_type=_F32)
    a = (g * lax.logistic(g) * u).astype(_BF16)
    o_ref[...] = jnp.dot(a, wd_ref[0], preferred_element_type=_F32)


def _moe_call(block_expert, xp, WG, WU, WD):
    grid_spec = pltpu.PrefetchScalarGridSpec(
        num_scalar_prefetch=1,
        grid=(NB,),
        in_specs=[
            pl.BlockSpec((BT, D), lambda b, eid: (b, 0)),
            pl.BlockSpec((1, D, I), lambda b, eid: (eid[b], 0, 0)),
            pl.BlockSpec((1, D, I), lambda b, eid: (eid[b], 0, 0)),
            pl.BlockSpec((1, I, D), lambda b, eid: (eid[b], 0, 0)),
        ],
        out_specs=pl.BlockSpec((BT, D), lambda b, eid: (b, 0)),
    )
    return pl.pallas_call(
        _moe_body,
        grid_spec=grid_spec,
        out_shape=jax.ShapeDtypeStruct((LP, D), _F32),
        compiler_params=pltpu.CompilerParams(
            dimension_semantics=("arbitrary",),
        ),
    )(block_expert, xp, WG, WU, WD)


def _sc_gather(table, idx):
    v_rows, d = table.shape
    n = idx.shape[0]
    info = plsc.get_sparse_core_info()
    nw = info.num_cores * info.num_subcores
    bpw = n // nw
    mesh = plsc.VectorSubcoreMesh(core_axis_name="c", subcore_axis_name="s")

    @functools.partial(
        pl.kernel,
        mesh=mesh,
        out_type=jax.ShapeDtypeStruct((n, d), _F32),
        scratch_types=[
            pltpu.VMEM((bpw,), jnp.int32),
            pltpu.VMEM((bpw, d), _F32),
            pltpu.SemaphoreType.DMA,
        ],
    )
    def k(table_hbm, idx_hbm, out_hbm, idx_v, rows_v, sem):
        wid = lax.axis_index("s") * info.num_cores + lax.axis_index("c")
        base = wid * bpw
        pltpu.sync_copy(idx_hbm.at[pl.ds(base, bpw)], idx_v)
        pltpu.async_copy(table_hbm.at[idx_v], rows_v, sem).wait()
        pltpu.sync_copy(rows_v, out_hbm.at[pl.ds(base, bpw)])

    return k(table, idx)


def kernel(positions, hidden_states, residual, gen_token_mask, rms1_w, rms2_w,
           wq, bq, wk, bk, wv, bv, wo, wg, wu, wd, gwg, gwu, gwd):
    h = hidden_states.reshape(L, D)
    r = residual.reshape(L, D)
    wqkv = jnp.concatenate([wq, wk, wv], axis=1).astype(_BF16)
    bqkv = jnp.concatenate([bq, bk, bv])[None, :]

    res1, qkv = _qkv_call(h, r, rms1_w[None, :], wqkv, bqkv)

    qkv3 = qkv.reshape(L, 3, H, HD).transpose(1, 2, 0, 3)

    pos = positions.reshape(-1).astype(_F32)
    inv_freq = 1.0 / (THETA ** (jnp.arange(HALF, dtype=_F32) / HALF))
    freqs = pos[:, None] * inv_freq[None, :]
    cos = jnp.cos(freqs)
    sin = jnp.sin(freqs)

    attn = _attn_call(qkv3[0], qkv3[1], qkv3[2], cos, sin)
    attn2 = attn.transpose(1, 0, 2).reshape(L, D)
    res2, h2 = _out_call(attn2, wo.astype(_BF16), res1, rms2_w[None, :])

    mi = gen_token_mask.reshape(-1).astype(jnp.int32)
    cg_cum = jnp.cumsum(mi)
    cu_cum = jnp.cumsum(1 - mi)
    cu = cu_cum[-1]
    nbu = (cu + BT - 1) // BT
    gen_start = nbu * BT
    dest = jnp.where(mi == 1, gen_start + cg_cum - 1, cu_cum - 1).astype(jnp.int32)
    perm = jnp.zeros((LP,), jnp.int32).at[dest].set(
        jnp.arange(L, dtype=jnp.int32))
    block_expert = (jnp.arange(NB, dtype=jnp.int32) >= nbu).astype(jnp.int32)

    xp = _sc_gather(h2, perm)
    WG = jnp.stack([wg, gwg]).astype(_BF16)
    WU = jnp.stack([wu, gwu]).astype(_BF16)
    WD = jnp.stack([wd, gwd]).astype(_BF16)
    mo = _moe_call(block_expert, xp, WG, WU, WD)
    out = _sc_gather(mo, dest)

    return (out.reshape(B, L, D), res2.reshape(B, L, D))
